# Initial kernel scaffold; baseline (speedup 1.0000x reference)
import jax
import jax.numpy as jnp
from jax.experimental import pallas as pl


def kernel(x, W):
    raise NotImplementedError("write your pallas kernel here")


# speedup vs baseline: 1.1965x; 1.1965x over previous
import functools

import jax
import jax.numpy as jnp
from jax.experimental import pallas as pl

LATENT_DIM = 32
CODEBOOK_SIZE = 512
COMMITMENT_WEIGHT = 0.25
N_TOKENS = 65536

BLOCK_N = 2048


def _vq_block_kernel(x_ref, w_ref, emb_ref, ids_ref, loss_ref):
    x = x_ref[...]
    w = w_ref[...]
    x2 = jnp.sum(x * x, axis=1, keepdims=True)
    w2 = jnp.sum(w * w, axis=1, keepdims=True).T
    xw = jax.lax.dot_general(x, w, (((1,), (1,)), ((), ())),
                             preferred_element_type=jnp.float32)
    dist = (x2 + w2) - 2.0 * xw
    min_d = jnp.min(dist, axis=1, keepdims=True)
    iota_k = jax.lax.broadcasted_iota(jnp.int32, dist.shape, 1)
    ids = jnp.min(jnp.where(dist == min_d, iota_k, CODEBOOK_SIZE),
                  axis=1, keepdims=True)
    onehot = (iota_k == ids).astype(jnp.float32)
    emb = jax.lax.dot_general(onehot, w, (((1,), (0,)), ((), ())),
                              preferred_element_type=jnp.float32,
                              precision=jax.lax.Precision.HIGHEST)
    emb_ref[...] = x + (emb - x)
    ids_ref[...] = ids
    diff = x - emb
    s = jnp.sum(diff * diff, axis=1, keepdims=True)
    loss_ref[...] = s + COMMITMENT_WEIGHT * s


@functools.partial(jax.jit, static_argnames=())
def kernel(x, W):
    n, d = x.shape
    k = W.shape[0]
    nb = n // BLOCK_N
    emb_out, ids2, loss2 = pl.pallas_call(
        _vq_block_kernel,
        grid=(nb,),
        in_specs=[
            pl.BlockSpec((BLOCK_N, d), lambda i: (i, 0)),
            pl.BlockSpec((k, d), lambda i: (0, 0)),
        ],
        out_specs=[
            pl.BlockSpec((BLOCK_N, d), lambda i: (i, 0)),
            pl.BlockSpec((BLOCK_N, 1), lambda i: (i, 0)),
            pl.BlockSpec((BLOCK_N, 1), lambda i: (i, 0)),
        ],
        out_shape=[
            jax.ShapeDtypeStruct((n, d), jnp.float32),
            jax.ShapeDtypeStruct((n, 1), jnp.int32),
            jax.ShapeDtypeStruct((n, 1), jnp.float32),
        ],
    )(x, W)
    return emb_out, ids2[:, 0], loss2[:, 0]


# speedup vs baseline: 1.5597x; 1.3035x over previous
import functools

import jax
import jax.numpy as jnp
from jax import lax
from jax.experimental import pallas as pl
from jax.experimental.pallas import tpu as pltpu
from jax.experimental.pallas import tpu_sc as plsc

LATENT_DIM = 32
CODEBOOK_SIZE = 512
COMMITMENT_WEIGHT = 0.25
N_TOKENS = 65536

BLOCK_N = 2048

_NW = 32
_BPW = N_TOKENS // _NW
_CHUNK = 128
_NCHUNK = _BPW // _CHUNK


def _argmin_loss_kernel(x_ref, w_ref, ids_ref, loss_ref):
    x = x_ref[...]
    w = w_ref[...]
    x2 = jnp.sum(x * x, axis=1, keepdims=True)
    w2 = jnp.sum(w * w, axis=1, keepdims=True).T
    xw = lax.dot_general(x, w, (((1,), (1,)), ((), ())),
                         preferred_element_type=jnp.float32)
    dist = (x2 + w2) - 2.0 * xw
    min_d = jnp.min(dist, axis=1, keepdims=True)
    iota_k = lax.broadcasted_iota(jnp.int32, dist.shape, 1)
    ids = jnp.min(jnp.where(dist == min_d, iota_k, CODEBOOK_SIZE),
                  axis=1, keepdims=True)
    ids_ref[...] = ids
    loss_ref[...] = min_d + COMMITMENT_WEIGHT * min_d


def _gather_kernel(table_hbm, idx_hbm, out_hbm, idx_v, rows_v, sem):
    wid = lax.axis_index("s") * 2 + lax.axis_index("c")
    base = wid * _BPW
    pltpu.sync_copy(idx_hbm.at[pl.ds(wid * _NCHUNK, _NCHUNK)], idx_v)
    copies = [
        pltpu.async_copy(table_hbm.at[idx_v.at[j]],
                         rows_v.at[pl.ds(j * _CHUNK, _CHUNK)], sem)
        for j in range(_NCHUNK)
    ]
    for c in copies:
        c.wait()
    pltpu.sync_copy(rows_v, out_hbm.at[pl.ds(base, _BPW)])


def _make_gather():
    mesh = plsc.VectorSubcoreMesh(core_axis_name="c", subcore_axis_name="s")
    return functools.partial(
        pl.kernel,
        mesh=mesh,
        compiler_params=pltpu.CompilerParams(use_tc_tiling_on_sc=False),
        out_type=jax.ShapeDtypeStruct((N_TOKENS, LATENT_DIM), jnp.float32),
        scratch_types=[
            pltpu.VMEM((_NCHUNK, _CHUNK), jnp.int32),
            pltpu.VMEM((_BPW, LATENT_DIM), jnp.float32),
            pltpu.SemaphoreType.DMA,
        ],
    )(_gather_kernel)


_gather = _make_gather()


@jax.jit
def kernel(x, W):
    n, d = x.shape
    k = W.shape[0]
    nb = n // BLOCK_N
    ids2, loss2 = pl.pallas_call(
        _argmin_loss_kernel,
        grid=(nb,),
        in_specs=[
            pl.BlockSpec((BLOCK_N, d), lambda i: (i, 0)),
            pl.BlockSpec((k, d), lambda i: (0, 0)),
        ],
        out_specs=[
            pl.BlockSpec((BLOCK_N, 1), lambda i: (i, 0)),
            pl.BlockSpec((BLOCK_N, 1), lambda i: (i, 0)),
        ],
        out_shape=[
            jax.ShapeDtypeStruct((n, 1), jnp.int32),
            jax.ShapeDtypeStruct((n, 1), jnp.float32),
        ],
    )(x, W)
    ids = ids2[:, 0]
    idx2d = ids.reshape(_NW * _NCHUNK, _CHUNK)
    emb_out = _gather(W, idx2d)
    return emb_out, ids, loss2[:, 0]


# speedup vs baseline: 1.7452x; 1.1189x over previous
import functools

import jax
import jax.numpy as jnp
from jax import lax
from jax.experimental import pallas as pl
from jax.experimental.pallas import tpu as pltpu
from jax.experimental.pallas import tpu_sc as plsc

LATENT_DIM = 32
CODEBOOK_SIZE = 512
COMMITMENT_WEIGHT = 0.25
N_TOKENS = 65536

BLOCK_N = 2048

_NW = 32
_BPW = N_TOKENS // _NW
_CHUNK = 128
_NCHUNK = _BPW // _CHUNK


def _argmin_loss_kernel(x_ref, w_ref, ids_ref, loss_ref):
    x = x_ref[...]
    w = w_ref[...]
    x2 = jnp.sum(x * x, axis=1, keepdims=True)
    w2 = jnp.sum(w * w, axis=1, keepdims=True).T
    xw = lax.dot_general(x, w, (((1,), (1,)), ((), ())),
                         preferred_element_type=jnp.float32)
    dist = (x2 + w2) - 2.0 * xw
    min_d = jnp.min(dist, axis=1, keepdims=True)
    iota_k = lax.broadcasted_iota(jnp.int32, dist.shape, 1)
    ids = jnp.min(jnp.where(dist == min_d, iota_k, CODEBOOK_SIZE),
                  axis=1, keepdims=True)
    ids_ref[...] = ids.reshape(BLOCK_N // 128, 128)
    loss = min_d + COMMITMENT_WEIGHT * min_d
    loss_ref[...] = loss.reshape(BLOCK_N // 128, 128)


def _gather_kernel(table_hbm, idx_hbm, out_hbm, idx_v, rows_v, sem):
    wid = lax.axis_index("s") * 2 + lax.axis_index("c")
    base = wid * _BPW
    pltpu.sync_copy(idx_hbm.at[pl.ds(wid * _NCHUNK, _NCHUNK)], idx_v)
    copies = [
        pltpu.async_copy(table_hbm.at[idx_v.at[j]],
                         rows_v.at[pl.ds(j * _CHUNK, _CHUNK)], sem)
        for j in range(_NCHUNK)
    ]
    for c in copies:
        c.wait()
    pltpu.sync_copy(rows_v, out_hbm.at[pl.ds(base, _BPW)])


def _make_gather():
    mesh = plsc.VectorSubcoreMesh(core_axis_name="c", subcore_axis_name="s")
    return functools.partial(
        pl.kernel,
        mesh=mesh,
        compiler_params=pltpu.CompilerParams(use_tc_tiling_on_sc=False),
        out_type=jax.ShapeDtypeStruct((N_TOKENS, LATENT_DIM), jnp.float32),
        scratch_types=[
            pltpu.VMEM((_NCHUNK, _CHUNK), jnp.int32),
            pltpu.VMEM((_BPW, LATENT_DIM), jnp.float32),
            pltpu.SemaphoreType.DMA,
        ],
    )(_gather_kernel)


_gather = _make_gather()


@jax.jit
def kernel(x, W):
    n, d = x.shape
    k = W.shape[0]
    nb = n // BLOCK_N
    ids2, loss2 = pl.pallas_call(
        _argmin_loss_kernel,
        grid=(nb,),
        in_specs=[
            pl.BlockSpec((BLOCK_N, d), lambda i: (i, 0)),
            pl.BlockSpec((k, d), lambda i: (0, 0)),
        ],
        out_specs=[
            pl.BlockSpec((BLOCK_N // 128, 128), lambda i: (i, 0)),
            pl.BlockSpec((BLOCK_N // 128, 128), lambda i: (i, 0)),
        ],
        out_shape=[
            jax.ShapeDtypeStruct((n // 128, 128), jnp.int32),
            jax.ShapeDtypeStruct((n // 128, 128), jnp.float32),
        ],
    )(x, W)
    emb_out = _gather(W, ids2)
    return emb_out, ids2.reshape(n), loss2.reshape(n)


# speedup vs baseline: 2.3489x; 1.3460x over previous
import functools

import jax
import jax.numpy as jnp
from jax import lax
from jax.experimental import pallas as pl
from jax.experimental.pallas import tpu as pltpu
from jax.experimental.pallas import tpu_sc as plsc

LATENT_DIM = 32
CODEBOOK_SIZE = 512
COMMITMENT_WEIGHT = 0.25
N_TOKENS = 65536

BLOCK_N = 2048

_NW = 32
_BPW = N_TOKENS // _NW
_L = 16


def _argmin_loss_kernel(xt_ref, wt_ref, ids_ref, loss_ref):
    xb = xt_ref[...]
    wb = wt_ref[...]
    x2 = jnp.sum(xb * xb, axis=0, keepdims=True)
    w2 = lax.dot_general(wb * wb, jnp.ones((LATENT_DIM, 1), jnp.float32),
                         (((0,), (0,)), ((), ())),
                         preferred_element_type=jnp.float32)
    m = lax.dot_general(wb, xb, (((0,), (0,)), ((), ())),
                        preferred_element_type=jnp.float32)
    dist = (x2 + w2) - 2.0 * m
    min_d = jnp.min(dist, axis=0, keepdims=True)
    iota_k = lax.broadcasted_iota(jnp.int32, dist.shape, 0)
    ids = jnp.min(jnp.where(dist == min_d, iota_k, CODEBOOK_SIZE),
                  axis=0, keepdims=True)
    ids_ref[...] = ids.reshape(BLOCK_N // 128, 128)
    loss = min_d + COMMITMENT_WEIGHT * min_d
    loss_ref[...] = loss.reshape(BLOCK_N // 128, 128)


_CHUNK = 128
_NCHUNK = _BPW // _CHUNK


def _gather_kernel(table_hbm, idx_hbm, out_hbm, idx_v, rows_v, sem):
    wid = lax.axis_index("s") * 2 + lax.axis_index("c")
    base = wid * _BPW
    pltpu.sync_copy(idx_hbm.at[pl.ds(wid * _NCHUNK, _NCHUNK)], idx_v)
    copies = [
        pltpu.async_copy(table_hbm.at[idx_v.at[j]],
                         rows_v.at[pl.ds(j * _CHUNK, _CHUNK)], sem)
        for j in range(_NCHUNK)
    ]
    for c in copies:
        c.wait()
    pltpu.sync_copy(rows_v, out_hbm.at[pl.ds(base, _BPW)])


def _make_gather():
    mesh = plsc.VectorSubcoreMesh(core_axis_name="c", subcore_axis_name="s")
    return functools.partial(
        pl.kernel,
        mesh=mesh,
        compiler_params=pltpu.CompilerParams(use_tc_tiling_on_sc=False),
        out_type=jax.ShapeDtypeStruct((N_TOKENS, LATENT_DIM), jnp.float32),
        scratch_types=[
            pltpu.VMEM((_NCHUNK, _CHUNK), jnp.int32),
            pltpu.VMEM((_BPW, LATENT_DIM), jnp.float32),
            pltpu.SemaphoreType.DMA,
        ],
    )(_gather_kernel)


_gather = _make_gather()


@jax.jit
def kernel(x, W):
    n, d = x.shape
    k = W.shape[0]
    nb = n // BLOCK_N
    xt = x.T
    wt = W.T
    ids2, loss2 = pl.pallas_call(
        _argmin_loss_kernel,
        grid=(nb,),
        in_specs=[
            pl.BlockSpec((d, BLOCK_N), lambda i: (0, i)),
            pl.BlockSpec((d, k), lambda i: (0, 0)),
        ],
        out_specs=[
            pl.BlockSpec((BLOCK_N // 128, 128), lambda i: (i, 0)),
            pl.BlockSpec((BLOCK_N // 128, 128), lambda i: (i, 0)),
        ],
        out_shape=[
            jax.ShapeDtypeStruct((n // 128, 128), jnp.int32),
            jax.ShapeDtypeStruct((n // 128, 128), jnp.float32),
        ],
    )(xt, wt)
    emb_out = _gather(W, ids2)
    return emb_out, ids2.reshape(n), loss2.reshape(n)
